# baseline (speedup 1.0000x reference)
import functools

import jax
import jax.numpy as jnp
from jax import lax
from jax.experimental import pallas as pl
from jax.experimental.pallas import tpu as pltpu
from jax.experimental.pallas import tpu_sc as plsc

KEY_CHUNK = 10000
SEQ_TILE = 512
PAD_B = 32


def _sc_gather_body(idx_hbm, val_hbm, cv_hbm, idx_v, rows_v, sem):
    wid = lax.axis_index("s") * 2 + lax.axis_index("c")

    @pl.when(wid == 0)
    def _():
        pltpu.sync_copy(idx_hbm, idx_v)
        pltpu.async_copy(val_hbm.at[idx_v], rows_v, sem).wait()
        pltpu.sync_copy(rows_v, cv_hbm)


def _query_body(x_ref, ew1_ref, eb1_ref, ew2_ref, eb2_ref, q_ref):
    xb = x_ref[0]
    S = xb.shape[0]
    ne = xb[:-1, :] != xb[1:, :]
    rowne = jnp.any(ne, axis=1, keepdims=True)
    j = lax.broadcasted_iota(jnp.int32, (S - 1, 1), 0) + 1
    cand = jnp.where(rowne, j, S + 7)
    first = jnp.min(cand)
    first = jnp.where(first >= S + 7, 0, first)
    first = jnp.where(first == 1, 0, first)
    pos = lax.broadcasted_iota(jnp.int32, (S, 1), 0)
    m = pos >= first
    cnt = (S - first).astype(jnp.float32)
    brow = jnp.sum(jnp.where(m, xb, 0.0), axis=0, keepdims=True) / cnt
    h = lax.dot_general(brow, ew1_ref[...], (((1,), (0,)), ((), ())),
                        preferred_element_type=jnp.float32) + eb1_ref[...]
    h = jnp.maximum(h, 0.0)
    q = lax.dot_general(h, ew2_ref[...], (((1,), (0,)), ((), ())),
                        preferred_element_type=jnp.float32) + eb2_ref[...]
    q_ref[0] = q


def _knn_body(k_ref, q_ref, bd2_ref, bidx_ref):
    ci = pl.program_id(0)
    keys = k_ref[...]
    q = q_ref[:, 0, :]
    chunk = keys.shape[0]
    n_total = pl.num_programs(0) * chunk
    ones = jnp.ones((1, keys.shape[1]), jnp.float32)
    kn_t = lax.dot_general(ones, keys * keys, (((1,), (1,)), ((), ())),
                           preferred_element_type=jnp.float32)
    qn = jnp.sum(q * q, axis=1)[:, None]
    cross_t = lax.dot_general(q, keys, (((1,), (1,)), ((), ())),
                              preferred_element_type=jnp.float32)
    d2 = jnp.maximum(kn_t + qn - 2.0 * cross_t, 0.0)
    mdt = jnp.min(d2, axis=1, keepdims=True)
    cols = lax.broadcasted_iota(jnp.int32, d2.shape, 1) + ci * chunk
    midxt = jnp.min(jnp.where(d2 == mdt, cols, n_total), axis=1,
                    keepdims=True)
    md = mdt.T
    midx = midxt.T

    @pl.when(ci == 0)
    def _():
        bd2_ref[...] = md
        bidx_ref[...] = midx

    @pl.when(ci > 0)
    def _():
        old = bd2_ref[...]
        better = md < old
        bd2_ref[...] = jnp.where(better, md, old)
        bidx_ref[...] = jnp.where(better, midx, bidx_ref[...])


def _out_body(idx_ref, x_ref, w_ref, b_ref, v_ref, e_ref, bd2_ref, o_ref):
    bb = pl.program_id(0)
    xt = x_ref[0].astype(jnp.bfloat16)
    yt = lax.dot_general(xt, w_ref[...], (((1,), (1,)), ((), ())),
                         preferred_element_type=jnp.float32) + b_ref[...]
    dist = jnp.sqrt(jnp.maximum(bd2_ref[0, bb], 0.0))
    lane = idx_ref[bb] % 128
    liota = lax.broadcasted_iota(jnp.int32, (1, 128), 1)
    eps1 = jnp.sum(jnp.where(liota == lane, e_ref[...][None, :], 0.0),
                   axis=1, keepdims=True)
    cond1 = dist <= eps1
    riota = lax.broadcasted_iota(jnp.int32, (8, 1), 0)
    vrow = jnp.sum(jnp.where(riota == bb, v_ref[...], 0.0),
                   axis=0, keepdims=True)
    o_ref[0] = jnp.where(cond1, vrow, yt)


def kernel(x, W, b, ew1, eb1, ew2, eb2, keys_store, values, epsilons):
    B, S, D = x.shape
    ENC = ew1.shape[1]
    N = keys_store.shape[0]
    n_chunks = N // KEY_CHUNK
    assert n_chunks * KEY_CHUNK == N

    query = pl.pallas_call(
        _query_body,
        grid=(B,),
        in_specs=[
            pl.BlockSpec((1, S, D), lambda i: (i, 0, 0)),
            pl.BlockSpec((D, ENC), lambda i: (0, 0)),
            pl.BlockSpec((1, ENC), lambda i: (0, 0)),
            pl.BlockSpec((ENC, ENC), lambda i: (0, 0)),
            pl.BlockSpec((1, ENC), lambda i: (0, 0)),
        ],
        out_specs=pl.BlockSpec((1, 1, ENC), lambda i: (i, 0, 0)),
        out_shape=jax.ShapeDtypeStruct((B, 1, ENC), jnp.float32),
    )(x, ew1, eb1.reshape(1, ENC), ew2, eb2.reshape(1, ENC))

    bd2, bidx = pl.pallas_call(
        _knn_body,
        grid=(n_chunks,),
        in_specs=[
            pl.BlockSpec((KEY_CHUNK, ENC), lambda i: (i, 0)),
            pl.BlockSpec((B, 1, ENC), lambda i: (0, 0, 0)),
        ],
        out_specs=[
            pl.BlockSpec((1, B), lambda i: (0, 0)),
            pl.BlockSpec((1, B), lambda i: (0, 0)),
        ],
        out_shape=[
            jax.ShapeDtypeStruct((1, B), jnp.float32),
            jax.ShapeDtypeStruct((1, B), jnp.int32),
        ],
    )(keys_store, query)

    idx = bidx.reshape(B)
    idxp = jnp.pad(idx, (0, PAD_B - B))

    sc_gather = functools.partial(
        pl.kernel,
        out_type=jax.ShapeDtypeStruct((PAD_B, D), jnp.float32),
        mesh=plsc.VectorSubcoreMesh(core_axis_name="c", subcore_axis_name="s"),
        scratch_types=[
            pltpu.VMEM((PAD_B,), jnp.int32),
            pltpu.VMEM((PAD_B, D), jnp.float32),
            pltpu.SemaphoreType.DMA,
        ],
    )(_sc_gather_body)
    cv = sc_gather(idxp, values)

    out = pl.pallas_call(
        _out_body,
        grid_spec=pltpu.PrefetchScalarGridSpec(
            num_scalar_prefetch=1,
            grid=(B, S // SEQ_TILE),
            in_specs=[
                pl.BlockSpec((1, SEQ_TILE, D), lambda bb, ss, idx: (bb, ss, 0)),
                pl.BlockSpec((D, D), lambda bb, ss, idx: (0, 0)),
                pl.BlockSpec((1, D), lambda bb, ss, idx: (0, 0)),
                pl.BlockSpec((8, D), lambda bb, ss, idx: (0, 0)),
                pl.BlockSpec((128,), lambda bb, ss, idx: (idx[bb] // 128,)),
                pl.BlockSpec(memory_space=pltpu.SMEM),
            ],
            out_specs=pl.BlockSpec((1, SEQ_TILE, D), lambda bb, ss, idx: (bb, ss, 0)),
        ),
        out_shape=jax.ShapeDtypeStruct((B, S, D), jnp.float32),
    )(idx, x, W.astype(jnp.bfloat16), b.reshape(1, D), cv, epsilons, bd2)
    return out


# speedup vs baseline: 1.1978x; 1.1978x over previous
import jax
import jax.numpy as jnp
from jax import lax
from jax.experimental import pallas as pl
from jax.experimental.pallas import tpu as pltpu

KEY_CHUNK = 10000
SEQ_TILE = 512


def _query_body(x_ref, ew1_ref, eb1_ref, ew2_ref, eb2_ref, q_ref):
    xb = x_ref[0]
    S = xb.shape[0]
    ne = xb[:-1, :] != xb[1:, :]
    rowne = jnp.any(ne, axis=1, keepdims=True)
    j = lax.broadcasted_iota(jnp.int32, (S - 1, 1), 0) + 1
    cand = jnp.where(rowne, j, S + 7)
    first = jnp.min(cand)
    first = jnp.where(first >= S + 7, 0, first)
    first = jnp.where(first == 1, 0, first)
    pos = lax.broadcasted_iota(jnp.int32, (S, 1), 0)
    m = pos >= first
    cnt = (S - first).astype(jnp.float32)
    brow = jnp.sum(jnp.where(m, xb, 0.0), axis=0, keepdims=True) / cnt
    h = lax.dot_general(brow, ew1_ref[...], (((1,), (0,)), ((), ())),
                        preferred_element_type=jnp.float32) + eb1_ref[...]
    h = jnp.maximum(h, 0.0)
    q = lax.dot_general(h, ew2_ref[...], (((1,), (0,)), ((), ())),
                        preferred_element_type=jnp.float32) + eb2_ref[...]
    q_ref[0] = q


def _knn_body(k_ref, q_ref, bd2_ref, bidx_ref):
    ci = pl.program_id(0)
    keys = k_ref[...]
    q = q_ref[:, 0, :]
    chunk = keys.shape[0]
    n_total = pl.num_programs(0) * chunk
    ones = jnp.ones((1, keys.shape[1]), jnp.float32)
    kn_t = lax.dot_general(ones, keys * keys, (((1,), (1,)), ((), ())),
                           preferred_element_type=jnp.float32)
    qn = jnp.sum(q * q, axis=1)[:, None]
    cross_t = lax.dot_general(q, keys, (((1,), (1,)), ((), ())),
                              preferred_element_type=jnp.float32)
    d2 = jnp.maximum(kn_t + qn - 2.0 * cross_t, 0.0)
    mdt = jnp.min(d2, axis=1, keepdims=True)
    cols = lax.broadcasted_iota(jnp.int32, d2.shape, 1) + ci * chunk
    midxt = jnp.min(jnp.where(d2 == mdt, cols, n_total), axis=1,
                    keepdims=True)
    md = mdt.T
    midx = midxt.T

    @pl.when(ci == 0)
    def _():
        bd2_ref[...] = md
        bidx_ref[...] = midx

    @pl.when(ci > 0)
    def _():
        old = bd2_ref[...]
        better = md < old
        bd2_ref[...] = jnp.where(better, md, old)
        bidx_ref[...] = jnp.where(better, midx, bidx_ref[...])


def _out_body(idx_ref, x_ref, w_ref, b_ref, v_ref, e_ref, bd2_ref, o_ref):
    bb = pl.program_id(0)
    xt = x_ref[0].astype(jnp.bfloat16)
    yt = lax.dot_general(xt, w_ref[...], (((1,), (1,)), ((), ())),
                         preferred_element_type=jnp.float32) + b_ref[...]
    dist = jnp.sqrt(jnp.maximum(bd2_ref[0, bb], 0.0))
    lane = idx_ref[bb] % 128
    liota = lax.broadcasted_iota(jnp.int32, (1, 128), 1)
    eps1 = jnp.sum(jnp.where(liota == lane, e_ref[...][None, :], 0.0),
                   axis=1, keepdims=True)
    cond1 = dist <= eps1
    r8 = idx_ref[bb] % 8
    riota = lax.broadcasted_iota(jnp.int32, (8, 1), 0)
    vrow = jnp.sum(jnp.where(riota == r8, v_ref[...], 0.0),
                   axis=0, keepdims=True)
    o_ref[0] = jnp.where(cond1, vrow, yt)


def kernel(x, W, b, ew1, eb1, ew2, eb2, keys_store, values, epsilons):
    B, S, D = x.shape
    ENC = ew1.shape[1]
    N = keys_store.shape[0]
    n_chunks = N // KEY_CHUNK
    assert n_chunks * KEY_CHUNK == N

    query = pl.pallas_call(
        _query_body,
        grid=(B,),
        in_specs=[
            pl.BlockSpec((1, S, D), lambda i: (i, 0, 0)),
            pl.BlockSpec((D, ENC), lambda i: (0, 0)),
            pl.BlockSpec((1, ENC), lambda i: (0, 0)),
            pl.BlockSpec((ENC, ENC), lambda i: (0, 0)),
            pl.BlockSpec((1, ENC), lambda i: (0, 0)),
        ],
        out_specs=pl.BlockSpec((1, 1, ENC), lambda i: (i, 0, 0)),
        out_shape=jax.ShapeDtypeStruct((B, 1, ENC), jnp.float32),
    )(x, ew1, eb1.reshape(1, ENC), ew2, eb2.reshape(1, ENC))

    bd2, bidx = pl.pallas_call(
        _knn_body,
        grid=(n_chunks,),
        in_specs=[
            pl.BlockSpec((KEY_CHUNK, ENC), lambda i: (i, 0)),
            pl.BlockSpec((B, 1, ENC), lambda i: (0, 0, 0)),
        ],
        out_specs=[
            pl.BlockSpec((1, B), lambda i: (0, 0)),
            pl.BlockSpec((1, B), lambda i: (0, 0)),
        ],
        out_shape=[
            jax.ShapeDtypeStruct((1, B), jnp.float32),
            jax.ShapeDtypeStruct((1, B), jnp.int32),
        ],
    )(keys_store, query)

    idx = bidx.reshape(B)

    out = pl.pallas_call(
        _out_body,
        grid_spec=pltpu.PrefetchScalarGridSpec(
            num_scalar_prefetch=1,
            grid=(B, S // SEQ_TILE),
            in_specs=[
                pl.BlockSpec((1, SEQ_TILE, D), lambda bb, ss, idx: (bb, ss, 0)),
                pl.BlockSpec((D, D), lambda bb, ss, idx: (0, 0)),
                pl.BlockSpec((1, D), lambda bb, ss, idx: (0, 0)),
                pl.BlockSpec((8, D), lambda bb, ss, idx: (idx[bb] // 8, 0)),
                pl.BlockSpec((128,), lambda bb, ss, idx: (idx[bb] // 128,)),
                pl.BlockSpec(memory_space=pltpu.SMEM),
            ],
            out_specs=pl.BlockSpec((1, SEQ_TILE, D), lambda bb, ss, idx: (bb, ss, 0)),
        ),
        out_shape=jax.ShapeDtypeStruct((B, S, D), jnp.float32),
    )(idx, x, W.astype(jnp.bfloat16), b.reshape(1, D), values, epsilons, bd2)
    return out
